# baseline (speedup 1.0000x reference)
import functools

import jax
import jax.numpy as jnp
from jax import lax
from jax.experimental import pallas as pl
from jax.experimental.pallas import tpu as pltpu
from jax.experimental.pallas import tpu_sc as plsc

BATCH = 4096
SEQ = 50
VOCAB = 100000
EMBED = 64
SEQ_P = 56
EMB_P = 128

NC = 2
NS = 16
NW = NC * NS
B_PER_W = BATCH // NW
NBUF = 8
PF = 6


def _sc_gather(idx_grid, weight):
    mesh = plsc.VectorSubcoreMesh(core_axis_name="c", subcore_axis_name="s")

    @functools.partial(
        pl.kernel,
        mesh=mesh,
        out_type=jax.ShapeDtypeStruct((BATCH, SEQ_P, EMB_P), jnp.float32),
        scratch_types=[
            pltpu.VMEM((B_PER_W, SEQ), jnp.int32),
            pltpu.VMEM((NBUF, SEQ, EMBED), jnp.float32),
            [pltpu.SemaphoreType.DMA] * NBUF,
            [pltpu.SemaphoreType.DMA] * NBUF,
        ],
        compiler_params=pltpu.CompilerParams(use_tc_tiling_on_sc=False),
    )
    def k(idx_hbm, table_hbm, out_hbm, idx_v, rows_v, g_sems, o_sems):
        wid = lax.axis_index("s") * NC + lax.axis_index("c")
        base = wid * B_PER_W
        pltpu.sync_copy(idx_hbm.at[wid], idx_v)

        def out_dst(row):
            return out_hbm.at[row, pl.ds(0, SEQ), pl.ds(0, EMBED)]

        for j in range(PF):
            pltpu.async_copy(table_hbm.at[idx_v.at[j]], rows_v.at[j], g_sems[j])

        def body(g, carry):
            for b in range(NBUF):
                j = g * NBUF + b
                pltpu.make_async_copy(
                    table_hbm.at[idx_v.at[b]], rows_v.at[b], g_sems[b]
                ).wait()
                pltpu.async_copy(rows_v.at[b], out_dst(base + j), o_sems[b])
                bb = (b + PF) % NBUF

                @pl.when(jnp.logical_and(j + PF < B_PER_W, j + PF >= NBUF))
                def _():
                    pltpu.make_async_copy(
                        rows_v.at[bb], out_dst(base), o_sems[bb]
                    ).wait()

                @pl.when(j + PF < B_PER_W)
                def _():
                    pltpu.async_copy(
                        table_hbm.at[idx_v.at[j + PF]], rows_v.at[bb], g_sems[bb]
                    )
            return carry

        lax.fori_loop(0, B_PER_W // NBUF, body, 0)

        for b in range(NBUF):
            pltpu.make_async_copy(
                rows_v.at[b], out_dst(base), o_sems[b]
            ).wait()

    return k(idx_grid, weight)


VB = 1024
VGRID = -(-VOCAB // VB)


def _tc_transpose_pad(wt):
    def body(wt_ref, out_ref):
        blk = wt_ref[...]
        t = jnp.transpose(blk)
        out_ref[...] = jnp.concatenate(
            [t, jnp.zeros((VB, EMBED), jnp.float32)], axis=-1
        )

    return pl.pallas_call(
        body,
        grid=(VGRID,),
        in_specs=[pl.BlockSpec((EMBED, VB), lambda i: (0, i))],
        out_specs=pl.BlockSpec((VB, 2 * EMBED), lambda i: (i, 0)),
        out_shape=jax.ShapeDtypeStruct((VOCAB, 2 * EMBED), jnp.float32),
    )(wt)


def kernel(input_ids, weight):
    idx_grid = (input_ids.astype(jnp.int32) * 2).reshape(NW, B_PER_W, SEQ)
    table2 = _tc_transpose_pad(weight.T).reshape(2 * VOCAB, EMBED)
    out_p = _sc_gather(idx_grid, table2)
    return out_p[:, :SEQ, :EMBED]


# speedup vs baseline: 1.1390x; 1.1390x over previous
import functools

import jax
import jax.numpy as jnp
from jax import lax
from jax.experimental import pallas as pl
from jax.experimental.pallas import tpu as pltpu
from jax.experimental.pallas import tpu_sc as plsc

BATCH = 4096
SEQ = 50
VOCAB = 100000
EMBED = 64
SEQ_P = 56
EMB_P = 128

NC = 2
NS = 16
NW = NC * NS
B_PER_W = BATCH // NW
NBUF = 8
PF = 7


def _sc_gather(idx_grid, weight):
    mesh = plsc.VectorSubcoreMesh(core_axis_name="c", subcore_axis_name="s")

    @functools.partial(
        pl.kernel,
        mesh=mesh,
        out_type=jax.ShapeDtypeStruct((BATCH, SEQ_P, EMB_P), jnp.float32),
        scratch_types=[
            pltpu.VMEM((B_PER_W, SEQ), jnp.int32),
            pltpu.VMEM((NBUF, SEQ, EMBED), jnp.float32),
            [pltpu.SemaphoreType.DMA] * NBUF,
            [pltpu.SemaphoreType.DMA] * NBUF,
        ],
        compiler_params=pltpu.CompilerParams(use_tc_tiling_on_sc=False),
    )
    def k(idx_hbm, table_hbm, out_hbm, idx_v, rows_v, g_sems, o_sems):
        wid = lax.axis_index("s") * NC + lax.axis_index("c")
        base = wid * B_PER_W
        pltpu.sync_copy(idx_hbm.at[wid], idx_v)

        def out_dst(row):
            return out_hbm.at[row, pl.ds(0, SEQ), pl.ds(0, EMBED)]

        for j in range(PF):
            pltpu.async_copy(table_hbm.at[idx_v.at[j]], rows_v.at[j], g_sems[j])

        def body(g, carry):
            for b in range(NBUF):
                j = g * NBUF + b
                pltpu.make_async_copy(
                    table_hbm.at[idx_v.at[b]], rows_v.at[b], g_sems[b]
                ).wait()
                pltpu.async_copy(rows_v.at[b], out_dst(base + j), o_sems[b])
                bb = (b + PF) % NBUF

                @pl.when(jnp.logical_and(j + PF < B_PER_W, j + PF >= NBUF))
                def _():
                    pltpu.make_async_copy(
                        rows_v.at[bb], out_dst(base), o_sems[bb]
                    ).wait()

                @pl.when(j + PF < B_PER_W)
                def _():
                    pltpu.async_copy(
                        table_hbm.at[idx_v.at[j + PF]], rows_v.at[bb], g_sems[bb]
                    )
            return carry

        lax.fori_loop(0, B_PER_W // NBUF, body, 0)

        for b in range(NBUF):
            pltpu.make_async_copy(
                rows_v.at[b], out_dst(base), o_sems[b]
            ).wait()

    return k(idx_grid, weight)


def kernel(input_ids, weight):
    idx_grid = (input_ids.astype(jnp.int32) * 2).reshape(NW, B_PER_W, SEQ)
    table2 = jnp.pad(weight, ((0, 0), (0, EMBED))).reshape(2 * VOCAB, EMBED)
    out_p = _sc_gather(idx_grid, table2)
    return out_p[:, :SEQ, :EMBED]
